# baseline (speedup 1.0000x reference)
import functools

import jax
import jax.numpy as jnp
from jax import lax
from jax.experimental import pallas as pl
from jax.experimental.pallas import tpu as pltpu
from jax.experimental.pallas import tpu_sc as plsc

NC = 2
NS = 16
NW = NC * NS
CHUNK = 128
NB = 4
DEGW = 16
NCH0 = 32
NCH1 = 128


def _scatter_rows(n_pad, d):
    rpt = n_pad // NS
    nchm = max(NCH0, NCH1)
    mesh = plsc.VectorSubcoreMesh(
        core_axis_name="c", subcore_axis_name="s", num_cores=NC, num_subcores=NS
    )
    scratch = (
        [pltpu.VMEM((nchm, CHUNK), jnp.int32)] * 2
        + [pltpu.VMEM((CHUNK, d), jnp.float32) for _ in range(NB)]
        + [pltpu.VMEM_SHARED((n_pad, d), jnp.float32)]
        + [pltpu.SemaphoreType.DMA] * (2 * NB)
    )

    @functools.partial(
        pl.kernel,
        out_type=jax.ShapeDtypeStruct((NC, n_pad, d), jnp.float32),
        mesh=mesh,
        scratch_types=scratch,
        compiler_params=pltpu.CompilerParams(use_tc_tiling_on_sc=False),
    )
    def k(feat, srcs, dsts, zeros, out, sidx, didx, *rest):
        rows = rest[:NB]
        acc = rest[NB]
        gsem = rest[NB + 1 : NB + 1 + NB]
        ssem = rest[NB + 1 + NB : NB + 1 + 2 * NB]
        c = lax.axis_index("c")
        s = lax.axis_index("s")
        w = c * NS + s
        nch = jnp.where(c == 0, NCH0, NCH1)
        pltpu.sync_copy(srcs.at[w], sidx)
        pltpu.sync_copy(dsts.at[w], didx)
        pltpu.sync_copy(zeros, acc.at[pl.ds(s * rpt, rpt)])
        plsc.subcore_barrier()
        lead = NB // 2
        for b in range(lead):
            pltpu.async_copy(feat.at[sidx.at[b]], rows[b], gsem[b])

        @pl.loop(0, nch, step=NB)
        def _(g):
            for b in range(NB):
                jj = g + b
                bn = (b + lead) % NB
                pltpu.make_async_copy(feat.at[sidx.at[jj]], rows[b], gsem[b]).wait()
                pltpu.async_copy(rows[b], acc.at[didx.at[jj]], ssem[b], add=True)
                nxt = jj + lead

                @pl.when(nxt < nch)
                def _():
                    prev = nxt - NB

                    @pl.when(prev >= 0)
                    def _():
                        pltpu.make_async_copy(
                            rows[bn], acc.at[didx.at[bn]], ssem[bn]
                        ).wait()

                    pltpu.async_copy(feat.at[sidx.at[nxt]], rows[bn], gsem[bn])

        for b in range(NB):
            pltpu.make_async_copy(rows[b], acc.at[didx.at[b]], ssem[b]).wait()
        plsc.subcore_barrier()
        pltpu.sync_copy(acc.at[pl.ds(s * rpt, rpt)], out.at[c, pl.ds(s * rpt, rpt)])

    return k


def _deg_counts(n_pad):
    rpt = n_pad // NS
    nchm = max(NCH0, NCH1)
    mesh = plsc.VectorSubcoreMesh(
        core_axis_name="c", subcore_axis_name="s", num_cores=NC, num_subcores=NS
    )
    G = 8

    @functools.partial(
        pl.kernel,
        out_type=jax.ShapeDtypeStruct((NC, n_pad, DEGW), jnp.float32),
        mesh=mesh,
        scratch_types=[
            pltpu.VMEM((nchm, CHUNK), jnp.int32),
            pltpu.VMEM((CHUNK, DEGW), jnp.float32),
            pltpu.VMEM_SHARED((n_pad, DEGW), jnp.float32),
            pltpu.SemaphoreType.DMA,
        ],
        compiler_params=pltpu.CompilerParams(use_tc_tiling_on_sc=False),
    )
    def k(dsts, ones, zeros, out, didx, onev, acc, sem):
        c = lax.axis_index("c")
        s = lax.axis_index("s")
        w = c * NS + s
        nch = jnp.where(c == 0, NCH0, NCH1)
        pltpu.sync_copy(dsts.at[w], didx)
        pltpu.sync_copy(ones, onev)
        pltpu.sync_copy(zeros, acc.at[pl.ds(s * rpt, rpt)])
        plsc.subcore_barrier()

        @pl.loop(0, nch, step=G)
        def _(g0):
            for b in range(G):
                pltpu.async_copy(onev, acc.at[didx.at[g0 + b]], sem, add=True)
            for b in range(G):
                pltpu.make_async_copy(onev, acc.at[didx.at[g0 + b]], sem).wait()

        plsc.subcore_barrier()
        pltpu.sync_copy(acc.at[pl.ds(s * rpt, rpt)], out.at[c, pl.ds(s * rpt, rpt)])

    return k


def _layer1(n, dfeat, dhid, br):
    def body(x_ref, w_ref, degp_ref, s1_ref, dinv_ref):
        dp = degp_ref[...]
        deg = dp[0, :, 0] + dp[1, :, 0] + 1.0
        dinv = lax.rsqrt(deg)
        xw = jnp.dot(x_ref[...], w_ref[...], preferred_element_type=jnp.float32)
        s1_ref[...] = xw * dinv[:, None]
        dinv_ref[...] = dinv[:, None]

    return pl.pallas_call(
        body,
        grid=(n // br,),
        in_specs=[
            pl.BlockSpec((br, dfeat), lambda r: (r, 0)),
            pl.BlockSpec((dfeat, dhid), lambda r: (0, 0)),
            pl.BlockSpec((NC, br, DEGW), lambda r: (0, r, 0)),
        ],
        out_specs=[
            pl.BlockSpec((br, dhid), lambda r: (r, 0)),
            pl.BlockSpec((br, 1), lambda r: (r, 0)),
        ],
        out_shape=[
            jax.ShapeDtypeStruct((n, dhid), jnp.float32),
            jax.ShapeDtypeStruct((n, 1), jnp.float32),
        ],
    )


def _layer2(n, dhid, br):
    def body(p_ref, s_ref, dinv_ref, b_ref, w_ref, out_ref):
        p = p_ref[...]
        dinv = dinv_ref[...]
        t = (p[0] + p[1] + s_ref[...]) * dinv
        h = jnp.maximum(t + b_ref[...], 0.0)
        out_ref[...] = (
            jnp.dot(h, w_ref[...], preferred_element_type=jnp.float32) * dinv
        )

    return pl.pallas_call(
        body,
        grid=(n // br,),
        in_specs=[
            pl.BlockSpec((NC, br, dhid), lambda r: (0, r, 0)),
            pl.BlockSpec((br, dhid), lambda r: (r, 0)),
            pl.BlockSpec((br, 1), lambda r: (r, 0)),
            pl.BlockSpec((1, dhid), lambda r: (0, 0)),
            pl.BlockSpec((dhid, dhid), lambda r: (0, 0)),
        ],
        out_specs=pl.BlockSpec((br, dhid), lambda r: (r, 0)),
        out_shape=jax.ShapeDtypeStruct((n, dhid), jnp.float32),
    )


def _final(n, dhid, br):
    def body(p_ref, s_ref, dinv_ref, b_ref, out_ref):
        p = p_ref[...]
        out_ref[...] = (p[0] + p[1] + s_ref[...]) * dinv_ref[...] + b_ref[...]

    return pl.pallas_call(
        body,
        grid=(n // br,),
        in_specs=[
            pl.BlockSpec((NC, br, dhid), lambda r: (0, r, 0)),
            pl.BlockSpec((br, dhid), lambda r: (r, 0)),
            pl.BlockSpec((br, 1), lambda r: (r, 0)),
            pl.BlockSpec((1, dhid), lambda r: (0, 0)),
        ],
        out_specs=pl.BlockSpec((br, dhid), lambda r: (r, 0)),
        out_shape=jax.ShapeDtypeStruct((n, dhid), jnp.float32),
    )


def kernel(x, edge_index, y, W1, b1, W2, b2):
    n, dfeat = x.shape
    dhid = W1.shape[1]
    e = edge_index.shape[1]

    nchm = max(NCH0, NCH1)
    sizes = [NCH0 * CHUNK] * NS + [NCH1 * CHUNK] * NS
    e_pad = sum(sizes)
    assert e_pad >= e, "per-core chunk counts too small for edge count"
    n_pad = -(-(n + 1) // (NS * 8)) * (NS * 8)

    def tile_layout(idx, fill):
        flat = jnp.concatenate([idx, jnp.full((e_pad - e,), fill, jnp.int32)])
        offs = [0]
        for sz in sizes:
            offs.append(offs[-1] + sz)
        rows = [
            jnp.pad(flat[offs[i] : offs[i + 1]], (0, nchm * CHUNK - sizes[i]))
            for i in range(NW)
        ]
        return jnp.stack(rows).reshape(NW, nchm, CHUNK)

    srcp = tile_layout(edge_index[0].astype(jnp.int32), 0)
    dstp = tile_layout(edge_index[1].astype(jnp.int32), n)

    zeros_d = jnp.zeros((n_pad // NS, dhid), jnp.float32)
    zeros_g = jnp.zeros((n_pad // NS, DEGW), jnp.float32)
    ones_g = jnp.ones((CHUNK, DEGW), jnp.float32)

    br = 2000
    degp = _deg_counts(n_pad)(dstp, ones_g, zeros_g)
    s1, dinv = _layer1(n, dfeat, dhid, br)(x, W1, degp)
    p1 = _scatter_rows(n_pad, dhid)(s1, srcp, dstp, zeros_d)
    s2 = _layer2(n, dhid, br)(p1, s1, dinv, b1.reshape(1, dhid), W2)
    p2 = _scatter_rows(n_pad, dhid)(s2, srcp, dstp, zeros_d)
    embed = _final(n, dhid, br)(p2, s2, dinv, b2.reshape(1, dhid))
    return embed


# speedup vs baseline: 1.1538x; 1.1538x over previous
import functools

import jax
import jax.numpy as jnp
from jax import lax
from jax.experimental import pallas as pl
from jax.experimental.pallas import tpu as pltpu
from jax.experimental.pallas import tpu_sc as plsc

NC = 2
NS = 16
NW = NC * NS
CHUNK = 128
NB = 4
DEGW = 16
NCH0 = 128
NCH1 = 32


def _scatter_rows(n_pad, d):
    rpt = n_pad // NS
    nchm = max(NCH0, NCH1)
    mesh = plsc.VectorSubcoreMesh(
        core_axis_name="c", subcore_axis_name="s", num_cores=NC, num_subcores=NS
    )
    scratch = (
        [pltpu.VMEM((nchm, CHUNK), jnp.int32)] * 2
        + [pltpu.VMEM((CHUNK, d), jnp.float32) for _ in range(NB)]
        + [pltpu.VMEM_SHARED((n_pad, d), jnp.float32)]
        + [pltpu.SemaphoreType.DMA] * (2 * NB)
    )

    @functools.partial(
        pl.kernel,
        out_type=jax.ShapeDtypeStruct((NC, n_pad, d), jnp.float32),
        mesh=mesh,
        scratch_types=scratch,
        compiler_params=pltpu.CompilerParams(use_tc_tiling_on_sc=False),
    )
    def k(feat, srcs, dsts, zeros, out, sidx, didx, *rest):
        rows = rest[:NB]
        acc = rest[NB]
        gsem = rest[NB + 1 : NB + 1 + NB]
        ssem = rest[NB + 1 + NB : NB + 1 + 2 * NB]
        c = lax.axis_index("c")
        s = lax.axis_index("s")
        w = c * NS + s
        nch = jnp.where(c == 0, NCH0, NCH1)
        pltpu.sync_copy(srcs.at[w], sidx)
        pltpu.sync_copy(dsts.at[w], didx)
        pltpu.sync_copy(zeros, acc.at[pl.ds(s * rpt, rpt)])
        plsc.subcore_barrier()
        lead = NB // 2
        for b in range(lead):
            pltpu.async_copy(feat.at[sidx.at[b]], rows[b], gsem[b])

        @pl.loop(0, nch, step=NB)
        def _(g):
            for b in range(NB):
                jj = g + b
                bn = (b + lead) % NB
                pltpu.make_async_copy(feat.at[sidx.at[jj]], rows[b], gsem[b]).wait()
                pltpu.async_copy(rows[b], acc.at[didx.at[jj]], ssem[b], add=True)
                nxt = jj + lead

                @pl.when(nxt < nch)
                def _():
                    prev = nxt - NB

                    @pl.when(prev >= 0)
                    def _():
                        pltpu.make_async_copy(
                            rows[bn], acc.at[didx.at[bn]], ssem[bn]
                        ).wait()

                    pltpu.async_copy(feat.at[sidx.at[nxt]], rows[bn], gsem[bn])

        for b in range(NB):
            pltpu.make_async_copy(rows[b], acc.at[didx.at[b]], ssem[b]).wait()
        plsc.subcore_barrier()
        pltpu.sync_copy(acc.at[pl.ds(s * rpt, rpt)], out.at[c, pl.ds(s * rpt, rpt)])

    return k


def _deg_counts(n_pad):
    rpt = n_pad // NS
    nchm = max(NCH0, NCH1)
    mesh = plsc.VectorSubcoreMesh(
        core_axis_name="c", subcore_axis_name="s", num_cores=NC, num_subcores=NS
    )
    G = 8

    @functools.partial(
        pl.kernel,
        out_type=jax.ShapeDtypeStruct((NC, n_pad, DEGW), jnp.float32),
        mesh=mesh,
        scratch_types=[
            pltpu.VMEM((nchm, CHUNK), jnp.int32),
            pltpu.VMEM((CHUNK, DEGW), jnp.float32),
            pltpu.VMEM_SHARED((n_pad, DEGW), jnp.float32),
            pltpu.SemaphoreType.DMA,
        ],
        compiler_params=pltpu.CompilerParams(use_tc_tiling_on_sc=False),
    )
    def k(dsts, ones, zeros, out, didx, onev, acc, sem):
        c = lax.axis_index("c")
        s = lax.axis_index("s")
        w = c * NS + s
        nch = jnp.where(c == 0, NCH0, NCH1)
        pltpu.sync_copy(dsts.at[w], didx)
        pltpu.sync_copy(ones, onev)
        pltpu.sync_copy(zeros, acc.at[pl.ds(s * rpt, rpt)])
        plsc.subcore_barrier()

        @pl.loop(0, nch, step=G)
        def _(g0):
            for b in range(G):
                pltpu.async_copy(onev, acc.at[didx.at[g0 + b]], sem, add=True)
            for b in range(G):
                pltpu.make_async_copy(onev, acc.at[didx.at[g0 + b]], sem).wait()

        plsc.subcore_barrier()
        pltpu.sync_copy(acc.at[pl.ds(s * rpt, rpt)], out.at[c, pl.ds(s * rpt, rpt)])

    return k


def _layer1(n, dfeat, dhid, br):
    def body(x_ref, w_ref, degp_ref, s1_ref, dinv_ref):
        dp = degp_ref[...]
        deg = dp[0, :, 0] + dp[1, :, 0] + 1.0
        dinv = lax.rsqrt(deg)
        xw = jnp.dot(x_ref[...], w_ref[...], preferred_element_type=jnp.float32)
        s1_ref[...] = xw * dinv[:, None]
        dinv_ref[...] = dinv[:, None]

    return pl.pallas_call(
        body,
        grid=(n // br,),
        in_specs=[
            pl.BlockSpec((br, dfeat), lambda r: (r, 0)),
            pl.BlockSpec((dfeat, dhid), lambda r: (0, 0)),
            pl.BlockSpec((NC, br, DEGW), lambda r: (0, r, 0)),
        ],
        out_specs=[
            pl.BlockSpec((br, dhid), lambda r: (r, 0)),
            pl.BlockSpec((br, 1), lambda r: (r, 0)),
        ],
        out_shape=[
            jax.ShapeDtypeStruct((n, dhid), jnp.float32),
            jax.ShapeDtypeStruct((n, 1), jnp.float32),
        ],
    )


def _layer2(n, dhid, br):
    def body(p_ref, s_ref, dinv_ref, b_ref, w_ref, out_ref):
        p = p_ref[...]
        dinv = dinv_ref[...]
        t = (p[0] + p[1] + s_ref[...]) * dinv
        h = jnp.maximum(t + b_ref[...], 0.0)
        out_ref[...] = (
            jnp.dot(h, w_ref[...], preferred_element_type=jnp.float32) * dinv
        )

    return pl.pallas_call(
        body,
        grid=(n // br,),
        in_specs=[
            pl.BlockSpec((NC, br, dhid), lambda r: (0, r, 0)),
            pl.BlockSpec((br, dhid), lambda r: (r, 0)),
            pl.BlockSpec((br, 1), lambda r: (r, 0)),
            pl.BlockSpec((1, dhid), lambda r: (0, 0)),
            pl.BlockSpec((dhid, dhid), lambda r: (0, 0)),
        ],
        out_specs=pl.BlockSpec((br, dhid), lambda r: (r, 0)),
        out_shape=jax.ShapeDtypeStruct((n, dhid), jnp.float32),
    )


def _final(n, dhid, br):
    def body(p_ref, s_ref, dinv_ref, b_ref, out_ref):
        p = p_ref[...]
        out_ref[...] = (p[0] + p[1] + s_ref[...]) * dinv_ref[...] + b_ref[...]

    return pl.pallas_call(
        body,
        grid=(n // br,),
        in_specs=[
            pl.BlockSpec((NC, br, dhid), lambda r: (0, r, 0)),
            pl.BlockSpec((br, dhid), lambda r: (r, 0)),
            pl.BlockSpec((br, 1), lambda r: (r, 0)),
            pl.BlockSpec((1, dhid), lambda r: (0, 0)),
        ],
        out_specs=pl.BlockSpec((br, dhid), lambda r: (r, 0)),
        out_shape=jax.ShapeDtypeStruct((n, dhid), jnp.float32),
    )


def kernel(x, edge_index, y, W1, b1, W2, b2):
    n, dfeat = x.shape
    dhid = W1.shape[1]
    e = edge_index.shape[1]

    nchm = max(NCH0, NCH1)
    sizes = [NCH0 * CHUNK] * NS + [NCH1 * CHUNK] * NS
    e_pad = sum(sizes)
    assert e_pad >= e, "per-core chunk counts too small for edge count"
    n_pad = -(-(n + 1) // (NS * 8)) * (NS * 8)

    def tile_layout(idx, fill):
        flat = jnp.concatenate([idx, jnp.full((e_pad - e,), fill, jnp.int32)])
        offs = [0]
        for sz in sizes:
            offs.append(offs[-1] + sz)
        rows = [
            jnp.pad(flat[offs[i] : offs[i + 1]], (0, nchm * CHUNK - sizes[i]))
            for i in range(NW)
        ]
        return jnp.stack(rows).reshape(NW, nchm, CHUNK)

    srcp = tile_layout(edge_index[0].astype(jnp.int32), 0)
    dstp = tile_layout(edge_index[1].astype(jnp.int32), n)

    zeros_d = jnp.zeros((n_pad // NS, dhid), jnp.float32)
    zeros_g = jnp.zeros((n_pad // NS, DEGW), jnp.float32)
    ones_g = jnp.ones((CHUNK, DEGW), jnp.float32)

    br = 2000
    degp = _deg_counts(n_pad)(dstp, ones_g, zeros_g)
    s1, dinv = _layer1(n, dfeat, dhid, br)(x, W1, degp)
    p1 = _scatter_rows(n_pad, dhid)(s1, srcp, dstp, zeros_d)
    s2 = _layer2(n, dhid, br)(p1, s1, dinv, b1.reshape(1, dhid), W2)
    p2 = _scatter_rows(n_pad, dhid)(s2, srcp, dstp, zeros_d)
    embed = _final(n, dhid, br)(p2, s2, dinv, b2.reshape(1, dhid))
    return embed


# speedup vs baseline: 1.1591x; 1.0046x over previous
import functools

import jax
import jax.numpy as jnp
from jax import lax
from jax.experimental import pallas as pl
from jax.experimental.pallas import tpu as pltpu
from jax.experimental.pallas import tpu_sc as plsc

NC = 1
NS = 16
NW = NC * NS
CHUNK = 128
NB = 4
DEGW = 16
NCH0 = 160
NCH1 = 160


def _scatter_rows(n_pad, d):
    rpt = n_pad // NS
    nchm = max(NCH0, NCH1)
    mesh = plsc.VectorSubcoreMesh(
        core_axis_name="c", subcore_axis_name="s", num_cores=NC, num_subcores=NS
    )
    scratch = (
        [pltpu.VMEM((nchm, CHUNK), jnp.int32)] * 2
        + [pltpu.VMEM((CHUNK, d), jnp.float32) for _ in range(NB)]
        + [pltpu.VMEM_SHARED((n_pad, d), jnp.float32)]
        + [pltpu.SemaphoreType.DMA] * (2 * NB)
    )

    @functools.partial(
        pl.kernel,
        out_type=jax.ShapeDtypeStruct((NC, n_pad, d), jnp.float32),
        mesh=mesh,
        scratch_types=scratch,
        compiler_params=pltpu.CompilerParams(use_tc_tiling_on_sc=False),
    )
    def k(feat, srcs, dsts, zeros, out, sidx, didx, *rest):
        rows = rest[:NB]
        acc = rest[NB]
        gsem = rest[NB + 1 : NB + 1 + NB]
        ssem = rest[NB + 1 + NB : NB + 1 + 2 * NB]
        c = lax.axis_index("c")
        s = lax.axis_index("s")
        w = c * NS + s
        nch = jnp.where(c == 0, NCH0, NCH1)
        pltpu.sync_copy(srcs.at[w], sidx)
        pltpu.sync_copy(dsts.at[w], didx)
        pltpu.sync_copy(zeros, acc.at[pl.ds(s * rpt, rpt)])
        plsc.subcore_barrier()
        lead = NB // 2
        for b in range(lead):
            pltpu.async_copy(feat.at[sidx.at[b]], rows[b], gsem[b])

        @pl.loop(0, nch, step=NB)
        def _(g):
            for b in range(NB):
                jj = g + b
                bn = (b + lead) % NB
                pltpu.make_async_copy(feat.at[sidx.at[jj]], rows[b], gsem[b]).wait()
                pltpu.async_copy(rows[b], acc.at[didx.at[jj]], ssem[b], add=True)
                nxt = jj + lead

                @pl.when(nxt < nch)
                def _():
                    prev = nxt - NB

                    @pl.when(prev >= 0)
                    def _():
                        pltpu.make_async_copy(
                            rows[bn], acc.at[didx.at[bn]], ssem[bn]
                        ).wait()

                    pltpu.async_copy(feat.at[sidx.at[nxt]], rows[bn], gsem[bn])

        for b in range(NB):
            pltpu.make_async_copy(rows[b], acc.at[didx.at[b]], ssem[b]).wait()
        plsc.subcore_barrier()
        pltpu.sync_copy(acc.at[pl.ds(s * rpt, rpt)], out.at[c, pl.ds(s * rpt, rpt)])

    return k


def _deg_counts(n_pad):
    rpt = n_pad // NS
    nchm = max(NCH0, NCH1)
    mesh = plsc.VectorSubcoreMesh(
        core_axis_name="c", subcore_axis_name="s", num_cores=NC, num_subcores=NS
    )
    G = 8

    @functools.partial(
        pl.kernel,
        out_type=jax.ShapeDtypeStruct((NC, n_pad, DEGW), jnp.float32),
        mesh=mesh,
        scratch_types=[
            pltpu.VMEM((nchm, CHUNK), jnp.int32),
            pltpu.VMEM((CHUNK, DEGW), jnp.float32),
            pltpu.VMEM_SHARED((n_pad, DEGW), jnp.float32),
            pltpu.SemaphoreType.DMA,
        ],
        compiler_params=pltpu.CompilerParams(use_tc_tiling_on_sc=False),
    )
    def k(dsts, ones, zeros, out, didx, onev, acc, sem):
        c = lax.axis_index("c")
        s = lax.axis_index("s")
        w = c * NS + s
        nch = jnp.where(c == 0, NCH0, NCH1)
        pltpu.sync_copy(dsts.at[w], didx)
        pltpu.sync_copy(ones, onev)
        pltpu.sync_copy(zeros, acc.at[pl.ds(s * rpt, rpt)])
        plsc.subcore_barrier()

        @pl.loop(0, nch, step=G)
        def _(g0):
            for b in range(G):
                pltpu.async_copy(onev, acc.at[didx.at[g0 + b]], sem, add=True)
            for b in range(G):
                pltpu.make_async_copy(onev, acc.at[didx.at[g0 + b]], sem).wait()

        plsc.subcore_barrier()
        pltpu.sync_copy(acc.at[pl.ds(s * rpt, rpt)], out.at[c, pl.ds(s * rpt, rpt)])

    return k


def _layer1(n, dfeat, dhid, br):
    def body(x_ref, w_ref, degp_ref, s1_ref, dinv_ref):
        dp = degp_ref[...]
        deg = jnp.sum(dp[:, :, 0], axis=0) + 1.0
        dinv = lax.rsqrt(deg)
        xw = jnp.dot(x_ref[...], w_ref[...], preferred_element_type=jnp.float32)
        s1_ref[...] = xw * dinv[:, None]
        dinv_ref[...] = dinv[:, None]

    return pl.pallas_call(
        body,
        grid=(n // br,),
        in_specs=[
            pl.BlockSpec((br, dfeat), lambda r: (r, 0)),
            pl.BlockSpec((dfeat, dhid), lambda r: (0, 0)),
            pl.BlockSpec((NC, br, DEGW), lambda r: (0, r, 0)),
        ],
        out_specs=[
            pl.BlockSpec((br, dhid), lambda r: (r, 0)),
            pl.BlockSpec((br, 1), lambda r: (r, 0)),
        ],
        out_shape=[
            jax.ShapeDtypeStruct((n, dhid), jnp.float32),
            jax.ShapeDtypeStruct((n, 1), jnp.float32),
        ],
    )


def _layer2(n, dhid, br):
    def body(p_ref, s_ref, dinv_ref, b_ref, w_ref, out_ref):
        p = p_ref[...]
        dinv = dinv_ref[...]
        t = (jnp.sum(p, axis=0) + s_ref[...]) * dinv
        h = jnp.maximum(t + b_ref[...], 0.0)
        out_ref[...] = (
            jnp.dot(h, w_ref[...], preferred_element_type=jnp.float32) * dinv
        )

    return pl.pallas_call(
        body,
        grid=(n // br,),
        in_specs=[
            pl.BlockSpec((NC, br, dhid), lambda r: (0, r, 0)),
            pl.BlockSpec((br, dhid), lambda r: (r, 0)),
            pl.BlockSpec((br, 1), lambda r: (r, 0)),
            pl.BlockSpec((1, dhid), lambda r: (0, 0)),
            pl.BlockSpec((dhid, dhid), lambda r: (0, 0)),
        ],
        out_specs=pl.BlockSpec((br, dhid), lambda r: (r, 0)),
        out_shape=jax.ShapeDtypeStruct((n, dhid), jnp.float32),
    )


def _final(n, dhid, br):
    def body(p_ref, s_ref, dinv_ref, b_ref, out_ref):
        p = p_ref[...]
        out_ref[...] = (jnp.sum(p, axis=0) + s_ref[...]) * dinv_ref[...] + b_ref[...]

    return pl.pallas_call(
        body,
        grid=(n // br,),
        in_specs=[
            pl.BlockSpec((NC, br, dhid), lambda r: (0, r, 0)),
            pl.BlockSpec((br, dhid), lambda r: (r, 0)),
            pl.BlockSpec((br, 1), lambda r: (r, 0)),
            pl.BlockSpec((1, dhid), lambda r: (0, 0)),
        ],
        out_specs=pl.BlockSpec((br, dhid), lambda r: (r, 0)),
        out_shape=jax.ShapeDtypeStruct((n, dhid), jnp.float32),
    )


def kernel(x, edge_index, y, W1, b1, W2, b2):
    n, dfeat = x.shape
    dhid = W1.shape[1]
    e = edge_index.shape[1]

    nchm = max(NCH0, NCH1)
    sizes = ([NCH0 * CHUNK] * NS + [NCH1 * CHUNK] * NS)[:NW]
    e_pad = sum(sizes)
    assert e_pad >= e, "per-core chunk counts too small for edge count"
    n_pad = -(-(n + 1) // (NS * 8)) * (NS * 8)

    def tile_layout(idx, fill):
        flat = jnp.concatenate([idx, jnp.full((e_pad - e,), fill, jnp.int32)])
        offs = [0]
        for sz in sizes:
            offs.append(offs[-1] + sz)
        rows = [
            jnp.pad(flat[offs[i] : offs[i + 1]], (0, nchm * CHUNK - sizes[i]))
            for i in range(NW)
        ]
        return jnp.stack(rows).reshape(NW, nchm, CHUNK)

    srcp = tile_layout(edge_index[0].astype(jnp.int32), 0)
    dstp = tile_layout(edge_index[1].astype(jnp.int32), n)

    zeros_d = jnp.zeros((n_pad // NS, dhid), jnp.float32)
    zeros_g = jnp.zeros((n_pad // NS, DEGW), jnp.float32)
    ones_g = jnp.ones((CHUNK, DEGW), jnp.float32)

    br = 2000
    degp = _deg_counts(n_pad)(dstp, ones_g, zeros_g)
    s1, dinv = _layer1(n, dfeat, dhid, br)(x, W1, degp)
    p1 = _scatter_rows(n_pad, dhid)(s1, srcp, dstp, zeros_d)
    s2 = _layer2(n, dhid, br)(p1, s1, dinv, b1.reshape(1, dhid), W2)
    p2 = _scatter_rows(n_pad, dhid)(s2, srcp, dstp, zeros_d)
    embed = _final(n, dhid, br)(p2, s2, dinv, b2.reshape(1, dhid))
    return embed


# speedup vs baseline: 1.4307x; 1.2344x over previous
import functools

import jax
import jax.numpy as jnp
from jax import lax
from jax.experimental import pallas as pl
from jax.experimental.pallas import tpu as pltpu
from jax.experimental.pallas import tpu_sc as plsc

NC = 1
NS = 16
NW = NC * NS
CHUNK = 128
NB = 4
DEGW = 16
NCH0 = 160
NCH1 = 160


def _scatter_rows(n_pad, d):
    rpt = n_pad // NS
    nchm = max(NCH0, NCH1)
    mesh = plsc.VectorSubcoreMesh(
        core_axis_name="c", subcore_axis_name="s", num_cores=NC, num_subcores=NS
    )
    scratch = (
        [pltpu.VMEM((nchm, CHUNK), jnp.int32)] * 2
        + [pltpu.VMEM((CHUNK, d), jnp.bfloat16) for _ in range(NB)]
        + [pltpu.VMEM_SHARED((n_pad, d), jnp.bfloat16)]
        + [pltpu.SemaphoreType.DMA] * (2 * NB)
    )

    @functools.partial(
        pl.kernel,
        out_type=jax.ShapeDtypeStruct((NC, n_pad, d), jnp.bfloat16),
        mesh=mesh,
        scratch_types=scratch,
        compiler_params=pltpu.CompilerParams(use_tc_tiling_on_sc=False),
    )
    def k(feat, srcs, dsts, zeros, out, sidx, didx, *rest):
        rows = rest[:NB]
        acc = rest[NB]
        gsem = rest[NB + 1 : NB + 1 + NB]
        ssem = rest[NB + 1 + NB : NB + 1 + 2 * NB]
        c = lax.axis_index("c")
        s = lax.axis_index("s")
        w = c * NS + s
        nch = jnp.where(c == 0, NCH0, NCH1)
        pltpu.sync_copy(srcs.at[w], sidx)
        pltpu.sync_copy(dsts.at[w], didx)
        pltpu.sync_copy(zeros, acc.at[pl.ds(s * rpt, rpt)])
        plsc.subcore_barrier()
        lead = NB // 2
        for b in range(lead):
            pltpu.async_copy(feat.at[sidx.at[b]], rows[b], gsem[b])

        @pl.loop(0, nch, step=NB)
        def _(g):
            for b in range(NB):
                jj = g + b
                bn = (b + lead) % NB
                pltpu.make_async_copy(feat.at[sidx.at[jj]], rows[b], gsem[b]).wait()
                pltpu.async_copy(rows[b], acc.at[didx.at[jj]], ssem[b], add=True)
                nxt = jj + lead

                @pl.when(nxt < nch)
                def _():
                    prev = nxt - NB

                    @pl.when(prev >= 0)
                    def _():
                        pltpu.make_async_copy(
                            rows[bn], acc.at[didx.at[bn]], ssem[bn]
                        ).wait()

                    pltpu.async_copy(feat.at[sidx.at[nxt]], rows[bn], gsem[bn])

        for b in range(NB):
            pltpu.make_async_copy(rows[b], acc.at[didx.at[b]], ssem[b]).wait()
        plsc.subcore_barrier()
        pltpu.sync_copy(acc.at[pl.ds(s * rpt, rpt)], out.at[c, pl.ds(s * rpt, rpt)])

    return k


def _deg_counts(n_pad):
    rpt = n_pad // NS
    nchm = max(NCH0, NCH1)
    mesh = plsc.VectorSubcoreMesh(
        core_axis_name="c", subcore_axis_name="s", num_cores=NC, num_subcores=NS
    )
    G = 8

    @functools.partial(
        pl.kernel,
        out_type=jax.ShapeDtypeStruct((NC, n_pad, DEGW), jnp.float32),
        mesh=mesh,
        scratch_types=[
            pltpu.VMEM((nchm, CHUNK), jnp.int32),
            pltpu.VMEM((CHUNK, DEGW), jnp.float32),
            pltpu.VMEM_SHARED((n_pad, DEGW), jnp.float32),
            pltpu.SemaphoreType.DMA,
        ],
        compiler_params=pltpu.CompilerParams(use_tc_tiling_on_sc=False),
    )
    def k(dsts, ones, zeros, out, didx, onev, acc, sem):
        c = lax.axis_index("c")
        s = lax.axis_index("s")
        w = c * NS + s
        nch = jnp.where(c == 0, NCH0, NCH1)
        pltpu.sync_copy(dsts.at[w], didx)
        pltpu.sync_copy(ones, onev)
        pltpu.sync_copy(zeros, acc.at[pl.ds(s * rpt, rpt)])
        plsc.subcore_barrier()

        @pl.loop(0, nch, step=G)
        def _(g0):
            for b in range(G):
                pltpu.async_copy(onev, acc.at[didx.at[g0 + b]], sem, add=True)
            for b in range(G):
                pltpu.make_async_copy(onev, acc.at[didx.at[g0 + b]], sem).wait()

        plsc.subcore_barrier()
        pltpu.sync_copy(acc.at[pl.ds(s * rpt, rpt)], out.at[c, pl.ds(s * rpt, rpt)])

    return k


def _layer1(n, dfeat, dhid, br):
    def body(x_ref, w_ref, degp_ref, s1_ref, s1b_ref, dinv_ref):
        dp = degp_ref[...]
        deg = jnp.sum(dp[:, :, 0], axis=0) + 1.0
        dinv = lax.rsqrt(deg)
        xw = jnp.dot(x_ref[...], w_ref[...], preferred_element_type=jnp.float32)
        s1 = xw * dinv[:, None]
        s1_ref[...] = s1
        s1b_ref[...] = s1.astype(jnp.bfloat16)
        dinv_ref[...] = dinv[:, None]

    return pl.pallas_call(
        body,
        grid=(n // br,),
        in_specs=[
            pl.BlockSpec((br, dfeat), lambda r: (r, 0)),
            pl.BlockSpec((dfeat, dhid), lambda r: (0, 0)),
            pl.BlockSpec((NC, br, DEGW), lambda r: (0, r, 0)),
        ],
        out_specs=[
            pl.BlockSpec((br, dhid), lambda r: (r, 0)),
            pl.BlockSpec((br, dhid), lambda r: (r, 0)),
            pl.BlockSpec((br, 1), lambda r: (r, 0)),
        ],
        out_shape=[
            jax.ShapeDtypeStruct((n, dhid), jnp.float32),
            jax.ShapeDtypeStruct((n, dhid), jnp.bfloat16),
            jax.ShapeDtypeStruct((n, 1), jnp.float32),
        ],
    )


def _layer2(n, dhid, br):
    def body(p_ref, s_ref, dinv_ref, b_ref, w_ref, out_ref, outb_ref):
        p = p_ref[...].astype(jnp.float32)
        dinv = dinv_ref[...]
        t = (jnp.sum(p, axis=0) + s_ref[...]) * dinv
        h = jnp.maximum(t + b_ref[...], 0.0)
        s2 = jnp.dot(h, w_ref[...], preferred_element_type=jnp.float32) * dinv
        out_ref[...] = s2
        outb_ref[...] = s2.astype(jnp.bfloat16)

    return pl.pallas_call(
        body,
        grid=(n // br,),
        in_specs=[
            pl.BlockSpec((NC, br, dhid), lambda r: (0, r, 0)),
            pl.BlockSpec((br, dhid), lambda r: (r, 0)),
            pl.BlockSpec((br, 1), lambda r: (r, 0)),
            pl.BlockSpec((1, dhid), lambda r: (0, 0)),
            pl.BlockSpec((dhid, dhid), lambda r: (0, 0)),
        ],
        out_specs=[
            pl.BlockSpec((br, dhid), lambda r: (r, 0)),
            pl.BlockSpec((br, dhid), lambda r: (r, 0)),
        ],
        out_shape=[
            jax.ShapeDtypeStruct((n, dhid), jnp.float32),
            jax.ShapeDtypeStruct((n, dhid), jnp.bfloat16),
        ],
    )


def _final(n, dhid, br):
    def body(p_ref, s_ref, dinv_ref, b_ref, out_ref):
        p = p_ref[...].astype(jnp.float32)
        out_ref[...] = (jnp.sum(p, axis=0) + s_ref[...]) * dinv_ref[...] + b_ref[...]

    return pl.pallas_call(
        body,
        grid=(n // br,),
        in_specs=[
            pl.BlockSpec((NC, br, dhid), lambda r: (0, r, 0)),
            pl.BlockSpec((br, dhid), lambda r: (r, 0)),
            pl.BlockSpec((br, 1), lambda r: (r, 0)),
            pl.BlockSpec((1, dhid), lambda r: (0, 0)),
        ],
        out_specs=pl.BlockSpec((br, dhid), lambda r: (r, 0)),
        out_shape=jax.ShapeDtypeStruct((n, dhid), jnp.float32),
    )


def kernel(x, edge_index, y, W1, b1, W2, b2):
    n, dfeat = x.shape
    dhid = W1.shape[1]
    e = edge_index.shape[1]

    nchm = max(NCH0, NCH1)
    sizes = ([NCH0 * CHUNK] * NS + [NCH1 * CHUNK] * NS)[:NW]
    e_pad = sum(sizes)
    assert e_pad >= e, "per-core chunk counts too small for edge count"
    n_pad = -(-(n + 1) // (NS * 8)) * (NS * 8)

    def tile_layout(idx, fill):
        flat = jnp.concatenate([idx, jnp.full((e_pad - e,), fill, jnp.int32)])
        offs = [0]
        for sz in sizes:
            offs.append(offs[-1] + sz)
        rows = [
            jnp.pad(flat[offs[i] : offs[i + 1]], (0, nchm * CHUNK - sizes[i]))
            for i in range(NW)
        ]
        return jnp.stack(rows).reshape(NW, nchm, CHUNK)

    srcp = tile_layout(edge_index[0].astype(jnp.int32), 0)
    dstp = tile_layout(edge_index[1].astype(jnp.int32), n)

    zeros_d = jnp.zeros((n_pad // NS, dhid), jnp.bfloat16)
    zeros_g = jnp.zeros((n_pad // NS, DEGW), jnp.float32)
    ones_g = jnp.ones((CHUNK, DEGW), jnp.float32)

    br = 2000
    degp = _deg_counts(n_pad)(dstp, ones_g, zeros_g)
    s1, s1b, dinv = _layer1(n, dfeat, dhid, br)(x, W1, degp)
    p1 = _scatter_rows(n_pad, dhid)(s1b, srcp, dstp, zeros_d)
    s2, s2b = _layer2(n, dhid, br)(p1, s1, dinv, b1.reshape(1, dhid), W2)
    p2 = _scatter_rows(n_pad, dhid)(s2b, srcp, dstp, zeros_d)
    embed = _final(n, dhid, br)(p2, s2, dinv, b2.reshape(1, dhid))
    return embed


# speedup vs baseline: 1.7778x; 1.2426x over previous
import functools

import jax
import jax.numpy as jnp
from jax import lax
from jax.experimental import pallas as pl
from jax.experimental.pallas import tpu as pltpu
from jax.experimental.pallas import tpu_sc as plsc

NC = 2
NS = 16
NW = NC * NS
CHUNK = 128
NB = 4
DEGW = 16
NCH0 = 80
NCH1 = 80


def _scatter_rows(n_pad, d):
    rpt = n_pad // NS
    nchm = max(NCH0, NCH1)
    mesh = plsc.VectorSubcoreMesh(
        core_axis_name="c", subcore_axis_name="s", num_cores=NC, num_subcores=NS
    )
    scratch = (
        [pltpu.VMEM((nchm, CHUNK), jnp.int32)] * 2
        + [pltpu.VMEM((CHUNK, d), jnp.bfloat16) for _ in range(NB)]
        + [pltpu.VMEM_SHARED((n_pad, d), jnp.bfloat16)]
        + [pltpu.SemaphoreType.DMA] * (2 * NB)
    )

    @functools.partial(
        pl.kernel,
        out_type=jax.ShapeDtypeStruct((NC, n_pad, d), jnp.bfloat16),
        mesh=mesh,
        scratch_types=scratch,
        compiler_params=pltpu.CompilerParams(use_tc_tiling_on_sc=False),
    )
    def k(feat, srcs, dsts, zeros, out, sidx, didx, *rest):
        rows = rest[:NB]
        acc = rest[NB]
        gsem = rest[NB + 1 : NB + 1 + NB]
        ssem = rest[NB + 1 + NB : NB + 1 + 2 * NB]
        c = lax.axis_index("c")
        s = lax.axis_index("s")
        w = c * NS + s
        nch = jnp.where(c == 0, NCH0, NCH1)
        pltpu.sync_copy(srcs.at[w], sidx)
        pltpu.sync_copy(dsts.at[w], didx)
        pltpu.sync_copy(zeros, acc.at[pl.ds(s * rpt, rpt)])
        plsc.subcore_barrier()
        lead = NB // 2
        for b in range(lead):
            pltpu.async_copy(feat.at[sidx.at[b]], rows[b], gsem[b])

        @pl.loop(0, nch, step=NB)
        def _(g):
            for b in range(NB):
                jj = g + b
                bn = (b + lead) % NB
                pltpu.make_async_copy(feat.at[sidx.at[jj]], rows[b], gsem[b]).wait()
                pltpu.async_copy(rows[b], acc.at[didx.at[jj]], ssem[b], add=True)
                nxt = jj + lead

                @pl.when(nxt < nch)
                def _():
                    prev = nxt - NB

                    @pl.when(prev >= 0)
                    def _():
                        pltpu.make_async_copy(
                            rows[bn], acc.at[didx.at[bn]], ssem[bn]
                        ).wait()

                    pltpu.async_copy(feat.at[sidx.at[nxt]], rows[bn], gsem[bn])

        for b in range(NB):
            pltpu.make_async_copy(rows[b], acc.at[didx.at[b]], ssem[b]).wait()
        plsc.subcore_barrier()
        pltpu.sync_copy(acc.at[pl.ds(s * rpt, rpt)], out.at[c, pl.ds(s * rpt, rpt)])

    return k


def _deg_counts(n_pad):
    rpt = n_pad // NS
    nchm = max(NCH0, NCH1)
    mesh = plsc.VectorSubcoreMesh(
        core_axis_name="c", subcore_axis_name="s", num_cores=NC, num_subcores=NS
    )
    G = 8

    @functools.partial(
        pl.kernel,
        out_type=jax.ShapeDtypeStruct((NC, n_pad, DEGW), jnp.float32),
        mesh=mesh,
        scratch_types=[
            pltpu.VMEM((nchm, CHUNK), jnp.int32),
            pltpu.VMEM((CHUNK, DEGW), jnp.float32),
            pltpu.VMEM_SHARED((n_pad, DEGW), jnp.float32),
            pltpu.SemaphoreType.DMA,
        ],
        compiler_params=pltpu.CompilerParams(use_tc_tiling_on_sc=False),
    )
    def k(dsts, ones, zeros, out, didx, onev, acc, sem):
        c = lax.axis_index("c")
        s = lax.axis_index("s")
        w = c * NS + s
        nch = jnp.where(c == 0, NCH0, NCH1)
        pltpu.sync_copy(dsts.at[w], didx)
        pltpu.sync_copy(ones, onev)
        pltpu.sync_copy(zeros, acc.at[pl.ds(s * rpt, rpt)])
        plsc.subcore_barrier()

        @pl.loop(0, nch, step=G)
        def _(g0):
            for b in range(G):
                pltpu.async_copy(onev, acc.at[didx.at[g0 + b]], sem, add=True)
            for b in range(G):
                pltpu.make_async_copy(onev, acc.at[didx.at[g0 + b]], sem).wait()

        plsc.subcore_barrier()
        pltpu.sync_copy(acc.at[pl.ds(s * rpt, rpt)], out.at[c, pl.ds(s * rpt, rpt)])

    return k


def _layer1(n, dfeat, dhid, br):
    def body(x_ref, w_ref, degp_ref, s1_ref, s1b_ref, dinv_ref):
        dp = degp_ref[...]
        deg = jnp.sum(dp[:, :, 0], axis=0) + 1.0
        dinv = lax.rsqrt(deg)
        xw = jnp.dot(x_ref[...], w_ref[...], preferred_element_type=jnp.float32)
        s1 = xw * dinv[:, None]
        s1_ref[...] = s1
        s1b_ref[...] = s1.astype(jnp.bfloat16)
        dinv_ref[...] = dinv[:, None]

    return pl.pallas_call(
        body,
        grid=(n // br,),
        in_specs=[
            pl.BlockSpec((br, dfeat), lambda r: (r, 0)),
            pl.BlockSpec((dfeat, dhid), lambda r: (0, 0)),
            pl.BlockSpec((NC, br, DEGW), lambda r: (0, r, 0)),
        ],
        out_specs=[
            pl.BlockSpec((br, dhid), lambda r: (r, 0)),
            pl.BlockSpec((br, dhid), lambda r: (r, 0)),
            pl.BlockSpec((br, 1), lambda r: (r, 0)),
        ],
        out_shape=[
            jax.ShapeDtypeStruct((n, dhid), jnp.float32),
            jax.ShapeDtypeStruct((n, dhid), jnp.bfloat16),
            jax.ShapeDtypeStruct((n, 1), jnp.float32),
        ],
    )


def _layer2(n, dhid, br):
    def body(p_ref, s_ref, dinv_ref, b_ref, w_ref, out_ref, outb_ref):
        p = p_ref[...].astype(jnp.float32)
        dinv = dinv_ref[...]
        t = (jnp.sum(p, axis=0) + s_ref[...]) * dinv
        h = jnp.maximum(t + b_ref[...], 0.0)
        s2 = jnp.dot(h, w_ref[...], preferred_element_type=jnp.float32) * dinv
        out_ref[...] = s2
        outb_ref[...] = s2.astype(jnp.bfloat16)

    return pl.pallas_call(
        body,
        grid=(n // br,),
        in_specs=[
            pl.BlockSpec((NC, br, dhid), lambda r: (0, r, 0)),
            pl.BlockSpec((br, dhid), lambda r: (r, 0)),
            pl.BlockSpec((br, 1), lambda r: (r, 0)),
            pl.BlockSpec((1, dhid), lambda r: (0, 0)),
            pl.BlockSpec((dhid, dhid), lambda r: (0, 0)),
        ],
        out_specs=[
            pl.BlockSpec((br, dhid), lambda r: (r, 0)),
            pl.BlockSpec((br, dhid), lambda r: (r, 0)),
        ],
        out_shape=[
            jax.ShapeDtypeStruct((n, dhid), jnp.float32),
            jax.ShapeDtypeStruct((n, dhid), jnp.bfloat16),
        ],
    )


def _final(n, dhid, br):
    def body(p_ref, s_ref, dinv_ref, b_ref, out_ref):
        p = p_ref[...].astype(jnp.float32)
        out_ref[...] = (jnp.sum(p, axis=0) + s_ref[...]) * dinv_ref[...] + b_ref[...]

    return pl.pallas_call(
        body,
        grid=(n // br,),
        in_specs=[
            pl.BlockSpec((NC, br, dhid), lambda r: (0, r, 0)),
            pl.BlockSpec((br, dhid), lambda r: (r, 0)),
            pl.BlockSpec((br, 1), lambda r: (r, 0)),
            pl.BlockSpec((1, dhid), lambda r: (0, 0)),
        ],
        out_specs=pl.BlockSpec((br, dhid), lambda r: (r, 0)),
        out_shape=jax.ShapeDtypeStruct((n, dhid), jnp.float32),
    )


def kernel(x, edge_index, y, W1, b1, W2, b2):
    n, dfeat = x.shape
    dhid = W1.shape[1]
    e = edge_index.shape[1]

    nchm = max(NCH0, NCH1)
    sizes = ([NCH0 * CHUNK] * NS + [NCH1 * CHUNK] * NS)[:NW]
    e_pad = sum(sizes)
    assert e_pad >= e, "per-core chunk counts too small for edge count"
    n_pad = -(-(n + 1) // (NS * 8)) * (NS * 8)

    def tile_layout(idx, fill):
        flat = jnp.concatenate([idx, jnp.full((e_pad - e,), fill, jnp.int32)])
        offs = [0]
        for sz in sizes:
            offs.append(offs[-1] + sz)
        rows = [
            jnp.pad(flat[offs[i] : offs[i + 1]], (0, nchm * CHUNK - sizes[i]))
            for i in range(NW)
        ]
        return jnp.stack(rows).reshape(NW, nchm, CHUNK)

    srcp = tile_layout(edge_index[0].astype(jnp.int32), 0)
    dstp = tile_layout(edge_index[1].astype(jnp.int32), n)

    zeros_d = jnp.zeros((n_pad // NS, dhid), jnp.bfloat16)
    zeros_g = jnp.zeros((n_pad // NS, DEGW), jnp.float32)
    ones_g = jnp.ones((CHUNK, DEGW), jnp.float32)

    br = 2000
    degp = _deg_counts(n_pad)(dstp, ones_g, zeros_g)
    s1, s1b, dinv = _layer1(n, dfeat, dhid, br)(x, W1, degp)
    p1 = _scatter_rows(n_pad, dhid)(s1b, srcp, dstp, zeros_d)
    s2, s2b = _layer2(n, dhid, br)(p1, s1, dinv, b1.reshape(1, dhid), W2)
    p2 = _scatter_rows(n_pad, dhid)(s2b, srcp, dstp, zeros_d)
    embed = _final(n, dhid, br)(p2, s2, dinv, b2.reshape(1, dhid))
    return embed
